# baseline (speedup 1.0000x reference)
import jax
import jax.numpy as jnp
from jax.experimental import pallas as pl
import jax.experimental.pallas.tpu as pltpu

USER_N = 8192
ITEM_N = 8192
HALF_N = USER_N // 2
EMB_D = 32
BJ = 512
NJ = ITEM_N // BJ


def _lightgcn_kernel(at_ref, ab_ref, eut_ref, eu_ref, ei_ref,
                     g1u, g2u, g3u, l1u, l2u, l3u,
                     g1i, g2i, g3i, l1i, l2i, l3i,
                     uw_acc, w1i_t, w3u_acc, u12_t):
    p = pl.program_id(0)
    j = pl.program_id(1)
    sl = pl.ds(j * BJ, BJ)
    D = EMB_D
    H = HALF_N

    @pl.when(p == 0)
    def _pass1():
        at = at_ref[...]
        ab = ab_ref[...]
        t1_t = (jax.lax.dot_general(
            eut_ref[:, :H], at, (((1,), (0,)), ((), ())),
            preferred_element_type=jnp.float32)
            + jax.lax.dot_general(
            eut_ref[:, H:], ab, (((1,), (0,)), ((), ())),
            preferred_element_type=jnp.float32))
        w1i_t[:, sl] = t1_t
        rhs = jnp.concatenate([ei_ref[...], t1_t.T], axis=1)
        pt = jax.lax.dot_general(
            at, rhs, (((1,), (0,)), ((), ())),
            preferred_element_type=jnp.float32)
        pb = jax.lax.dot_general(
            ab, rhs, (((1,), (0,)), ((), ())),
            preferred_element_type=jnp.float32)

        @pl.when(j == 0)
        def _():
            uw_acc[:H, :] = pt
            uw_acc[H:, :] = pb

        @pl.when(j > 0)
        def _():
            uw_acc[:H, :] += pt
            uw_acc[H:, :] += pb

    @pl.when((p == 1) & (j == 0))
    def _mid():
        u12_t[...] = uw_acc[...].T

    @pl.when(p == 1)
    def _pass2():
        at = at_ref[...]
        ab = ab_ref[...]
        s_t = (jax.lax.dot_general(
            u12_t[:, :H], at, (((1,), (0,)), ((), ())),
            preferred_element_type=jnp.float32)
            + jax.lax.dot_general(
            u12_t[:, H:], ab, (((1,), (0,)), ((), ())),
            preferred_element_type=jnp.float32))
        w1i = w1i_t[:, sl].T
        w2i = s_t[0:D, :].T
        w3i = s_t[D:2 * D, :].T
        ei = ei_ref[...]
        g1i[...] = w1i
        g2i[...] = w1i + w2i
        g3i[...] = w1i + 2.0 * w2i + w3i
        l1i[...] = ei + w1i
        l2i[...] = ei + 2.0 * w1i + w2i
        l3i[...] = ei + 3.0 * w1i + 3.0 * w2i + w3i
        w2i_stripe = w2i
        qt = jax.lax.dot_general(
            at, w2i_stripe, (((1,), (0,)), ((), ())),
            preferred_element_type=jnp.float32)
        qb = jax.lax.dot_general(
            ab, w2i_stripe, (((1,), (0,)), ((), ())),
            preferred_element_type=jnp.float32)

        @pl.when(j == 0)
        def _():
            w3u_acc[:H, :] = qt
            w3u_acc[H:, :] = qb

        @pl.when(j > 0)
        def _():
            w3u_acc[:H, :] += qt
            w3u_acc[H:, :] += qb

    @pl.when(p == 2)
    def _epilogue():
        w1u = uw_acc[sl, 0:D]
        w2u = uw_acc[sl, D:2 * D]
        w3u = w3u_acc[sl, :]
        eu = eu_ref[...]
        g1u[...] = w1u
        g2u[...] = w1u + w2u
        g3u[...] = w1u + 2.0 * w2u + w3u
        l1u[...] = eu + w1u
        l2u[...] = eu + 2.0 * w1u + w2u
        l3u[...] = eu + 3.0 * w1u + 3.0 * w2u + w3u


def _run(adj, e_u_t, e_u, e_i):
    D = EMB_D
    out_sd = jax.ShapeDtypeStruct((USER_N, D), jnp.float32)
    out_shape = [out_sd] * 12

    def top_map(p, j):
        return (0, jnp.where(p == 2, NJ - 1, j))

    def bot_map(p, j):
        return (1, jnp.where(p == 2, NJ - 1, j))

    def chunk_map(p, j):
        return (jnp.where(p == 2, j, 0), 0)

    def item_map(p, j):
        return (jnp.where(p == 1, j, jnp.where(p == 2, NJ - 1, 0)), 0)

    return pl.pallas_call(
        _lightgcn_kernel,
        grid=(3, NJ),
        in_specs=[
            pl.BlockSpec((HALF_N, BJ), top_map),
            pl.BlockSpec((HALF_N, BJ), bot_map),
            pl.BlockSpec((D, USER_N), lambda p, j: (0, 0)),
            pl.BlockSpec((BJ, D), chunk_map),
            pl.BlockSpec((BJ, D), lambda p, j: (j, 0)),
        ],
        out_specs=([pl.BlockSpec((BJ, D), chunk_map)] * 6
                   + [pl.BlockSpec((BJ, D), item_map)] * 6),
        out_shape=out_shape,
        scratch_shapes=[
            pltpu.VMEM((USER_N, 2 * D), jnp.float32),
            pltpu.VMEM((D, ITEM_N), jnp.float32),
            pltpu.VMEM((USER_N, D), jnp.float32),
            pltpu.VMEM((2 * D, USER_N), jnp.float32),
        ],
    )(adj, adj, e_u_t, e_u, e_i)


def kernel(adj, embeds):
    e_u = embeds[:USER_N]
    e_i = embeds[USER_N:]
    e_u_t = e_u.T
    (g1u, g2u, g3u, l1u, l2u, l3u,
     g1i, g2i, g3i, l1i, l2i, l3i) = _run(adj, e_u_t, e_u, e_i)
    lats = (embeds,
            jnp.concatenate([l1u, l1i], axis=0),
            jnp.concatenate([l2u, l2i], axis=0),
            jnp.concatenate([l3u, l3i], axis=0))
    gcn_lats = (embeds,
                jnp.concatenate([g1u, g1i], axis=0),
                jnp.concatenate([g2u, g2i], axis=0),
                jnp.concatenate([g3u, g3i], axis=0))
    return (lats, gcn_lats)


# speedup vs baseline: 1.0017x; 1.0017x over previous
import jax
import jax.numpy as jnp
from jax.experimental import pallas as pl
import jax.experimental.pallas.tpu as pltpu

USER_N = 8192
ITEM_N = 8192
HALF_N = USER_N // 2
EMB_D = 32
BJ = 512
NJ = ITEM_N // BJ


def _lightgcn_kernel(at_ref, ab_ref, eut_ref, eu_ref, ei_ref,
                     g1u, g2u, g3u, l1u, l2u, l3u,
                     g1i, g2i, g3i, l1i, l2i, l3i,
                     uw_acc, w1i_t, wi23_t, w3u_acc, u12_t):
    p = pl.program_id(0)
    j = pl.program_id(1)
    sl = pl.ds(j * BJ, BJ)
    D = EMB_D
    H = HALF_N

    @pl.when(p == 0)
    def _pass1():
        at = at_ref[...]
        ab = ab_ref[...]
        t1_t = (jax.lax.dot_general(
            eut_ref[:, :H], at, (((1,), (0,)), ((), ())),
            preferred_element_type=jnp.float32)
            + jax.lax.dot_general(
            eut_ref[:, H:], ab, (((1,), (0,)), ((), ())),
            preferred_element_type=jnp.float32))
        w1i_t[:, sl] = t1_t
        rhs = jnp.concatenate([ei_ref[...], t1_t.T], axis=1)
        pt = jax.lax.dot_general(
            at, rhs, (((1,), (0,)), ((), ())),
            preferred_element_type=jnp.float32)
        pb = jax.lax.dot_general(
            ab, rhs, (((1,), (0,)), ((), ())),
            preferred_element_type=jnp.float32)

        @pl.when(j == 0)
        def _():
            uw_acc[:H, :] = pt
            uw_acc[H:, :] = pb

        @pl.when(j > 0)
        def _():
            uw_acc[:H, :] += pt
            uw_acc[H:, :] += pb

    @pl.when((p == 1) & (j == 0))
    def _mid():
        u12_t[...] = uw_acc[...].T

    @pl.when(p == 1)
    def _pass2():
        at = at_ref[...]
        ab = ab_ref[...]
        s_t = (jax.lax.dot_general(
            u12_t[:, :H], at, (((1,), (0,)), ((), ())),
            preferred_element_type=jnp.float32)
            + jax.lax.dot_general(
            u12_t[:, H:], ab, (((1,), (0,)), ((), ())),
            preferred_element_type=jnp.float32))
        wi23_t[:, sl] = s_t
        w2i_stripe = s_t[0:D, :].T
        qt = jax.lax.dot_general(
            at, w2i_stripe, (((1,), (0,)), ((), ())),
            preferred_element_type=jnp.float32)
        qb = jax.lax.dot_general(
            ab, w2i_stripe, (((1,), (0,)), ((), ())),
            preferred_element_type=jnp.float32)

        @pl.when(j == 0)
        def _():
            w3u_acc[:H, :] = qt
            w3u_acc[H:, :] = qb

        @pl.when(j > 0)
        def _():
            w3u_acc[:H, :] += qt
            w3u_acc[H:, :] += qb

    @pl.when(p == 2)
    def _epilogue():
        w1u = uw_acc[sl, 0:D]
        w2u = uw_acc[sl, D:2 * D]
        w3u = w3u_acc[sl, :]
        eu = eu_ref[...]
        g1u[...] = w1u
        g2u[...] = w1u + w2u
        g3u[...] = w1u + 2.0 * w2u + w3u
        l1u[...] = eu + w1u
        l2u[...] = eu + 2.0 * w1u + w2u
        l3u[...] = eu + 3.0 * w1u + 3.0 * w2u + w3u

        w1i = w1i_t[:, sl].T
        w23 = wi23_t[:, sl].T
        w2i = w23[:, 0:D]
        w3i = w23[:, D:2 * D]
        ei = ei_ref[...]
        g1i[...] = w1i
        g2i[...] = w1i + w2i
        g3i[...] = w1i + 2.0 * w2i + w3i
        l1i[...] = ei + w1i
        l2i[...] = ei + 2.0 * w1i + w2i
        l3i[...] = ei + 3.0 * w1i + 3.0 * w2i + w3i


def _run(adj, e_u_t, e_u, e_i):
    D = EMB_D
    out_sd = jax.ShapeDtypeStruct((USER_N, D), jnp.float32)
    out_shape = [out_sd] * 12

    def top_map(p, j):
        return (0, jnp.where(p == 2, NJ - 1, j))

    def bot_map(p, j):
        return (1, jnp.where(p == 2, NJ - 1, j))

    def chunk_map(p, j):
        return (jnp.where(p == 2, j, 0), 0)

    return pl.pallas_call(
        _lightgcn_kernel,
        grid=(3, NJ),
        in_specs=[
            pl.BlockSpec((HALF_N, BJ), top_map),
            pl.BlockSpec((HALF_N, BJ), bot_map),
            pl.BlockSpec((D, USER_N), lambda p, j: (0, 0)),
            pl.BlockSpec((BJ, D), chunk_map),
            pl.BlockSpec((BJ, D), lambda p, j: (j, 0)),
        ],
        out_specs=[pl.BlockSpec((BJ, D), chunk_map)] * 12,
        out_shape=out_shape,
        scratch_shapes=[
            pltpu.VMEM((USER_N, 2 * D), jnp.float32),
            pltpu.VMEM((D, ITEM_N), jnp.float32),
            pltpu.VMEM((2 * D, ITEM_N), jnp.float32),
            pltpu.VMEM((USER_N, D), jnp.float32),
            pltpu.VMEM((2 * D, USER_N), jnp.float32),
        ],
    )(adj, adj, e_u_t, e_u, e_i)


def kernel(adj, embeds):
    e_u = embeds[:USER_N]
    e_i = embeds[USER_N:]
    e_u_t = e_u.T
    (g1u, g2u, g3u, l1u, l2u, l3u,
     g1i, g2i, g3i, l1i, l2i, l3i) = _run(adj, e_u_t, e_u, e_i)
    lats = (embeds,
            jnp.concatenate([l1u, l1i], axis=0),
            jnp.concatenate([l2u, l2i], axis=0),
            jnp.concatenate([l3u, l3i], axis=0))
    gcn_lats = (embeds,
                jnp.concatenate([g1u, g1i], axis=0),
                jnp.concatenate([g2u, g2i], axis=0),
                jnp.concatenate([g3u, g3i], axis=0))
    return (lats, gcn_lats)
